# baseline (speedup 1.0000x reference)
import functools

import jax
import jax.numpy as jnp
from jax import lax
from jax.experimental import pallas as pl
from jax.experimental.pallas import tpu as pltpu
from jax.experimental.pallas import tpu_sc as plsc

N = 10000
E = 160000
D = 256
H = 256
G = 64

NC = 2
NS = 16
NW = NC * NS
EPAD = 160256
EPW = EPAD // NW
NV = EPW // 16
MPAD = N + 80
PADDST = MPAD - 1
CAP = 512
CHUNK = 64


def _k1_body(x_ref, w_ref, asrc_ref, adst_ref, batch_ref,
             o_asrc, o_adst, o_mark, o_soh):
    f32 = jnp.float32
    i32 = jnp.int32
    x = x_ref[...]
    W = w_ref[...]
    wsrc = jnp.dot(W, asrc_ref[...], preferred_element_type=f32)
    wdst = jnp.dot(W, adst_ref[...], preferred_element_type=f32)
    a_src = jnp.dot(x, wsrc, preferred_element_type=f32)
    a_dst = jnp.dot(x, wdst, preferred_element_type=f32)
    o_asrc[...] = a_src

    batch = batch_ref[...]
    gcol = lax.broadcasted_iota(i32, (G, 1), 0)
    eqg = (gcol == batch).astype(f32)
    ones_col = jnp.ones((N, 1), f32)
    counts = jnp.dot(eqg, ones_col, preferred_element_type=f32)
    g_r = lax.broadcasted_iota(i32, (G, G), 0)
    g_c = lax.broadcasted_iota(i32, (G, G), 1)
    lt = (g_c <= g_r).astype(f32)
    cum = jnp.dot(lt, counts, preferred_element_type=f32)
    sel = cum.astype(i32) - 1
    selw = jnp.where(sel < 0, sel + N, sel)

    n2 = lax.broadcasted_iota(i32, (G, MPAD), 1)
    ohsel = (selw == n2).astype(f32)
    eqm = lax.dot_general(ohsel, ohsel, (((1,), (1,)), ((), ())),
                          preferred_element_type=f32)
    slt = (g_c < g_r).astype(f32)
    dup_before = jnp.dot(eqm * slt, jnp.ones((G, 1), f32),
                         preferred_element_type=f32)
    winner = dup_before < 0.5
    sel_mark = jnp.where(winner, selw, N + gcol)
    ohm = (sel_mark == n2).astype(f32)
    jp1 = (gcol + 1).astype(f32)
    markv = lax.dot_general(ohm, jp1, (((0,), (0,)), ((), ())),
                            preferred_element_type=f32)
    o_mark[...] = markv.astype(i32) - 1
    slot_p1 = jnp.dot(ohsel, markv, preferred_element_type=f32)
    slot_of_g = slot_p1.astype(i32) - 1
    o_soh[...] = (slot_of_g == g_c).astype(f32)
    o_adst[...] = jnp.dot(ohm[:, :N], a_dst, preferred_element_type=f32)


_k1 = pl.pallas_call(
    _k1_body,
    out_shape=(
        jax.ShapeDtypeStruct((N, 1), jnp.float32),
        jax.ShapeDtypeStruct((G, 1), jnp.float32),
        jax.ShapeDtypeStruct((MPAD, 1), jnp.int32),
        jax.ShapeDtypeStruct((G, G), jnp.float32),
    ),
)


def _k2_body(src_up, dst_up, src_dn, dst_dn, asrc_up, asrc_dn,
             adst_up, adst_dn, mark_up, mark_dn, x_up, x_dn,
             o_num_up, o_den_up, o_num_dn, o_den_dn,
             e_src, e_dst, t_asrc, t_mark, t_adst,
             hit_src, hit_slot, hit_e, idx_stage, rowbuf, numer_v, denom_v,
             sem):
    i32 = jnp.int32
    f32 = jnp.float32
    cid = lax.axis_index("c")
    sid = lax.axis_index("s")
    wid = sid * NC + cid
    base_e = wid * EPW
    iota16 = lax.iota(i32, 16)
    zf = jnp.zeros((16,), f32)
    zi = jnp.zeros((16,), i32)
    lane0 = iota16 == 0

    sides = (
        (src_up, dst_up, asrc_up, adst_up, mark_up, x_up, o_num_up, o_den_up),
        (src_dn, dst_dn, asrc_dn, adst_dn, mark_dn, x_dn, o_num_dn, o_den_dn),
    )
    for sidx, (srcR, dstR, asrcR, adstR, markR, xR, onumR, odenR) in enumerate(sides):
      with jax.named_scope(f"stage{sidx}"):
        pltpu.sync_copy(srcR.at[pl.ds(base_e, EPW)], e_src)
        pltpu.sync_copy(dstR.at[pl.ds(base_e, EPW)], e_dst)
        pltpu.sync_copy(asrcR, t_asrc)
        pltpu.sync_copy(markR, t_mark)
        pltpu.sync_copy(adstR, t_adst)

        def _znum(i, c):
            numer_v[pl.ds(i * 16, 16)] = zf
            return c
        lax.fori_loop(0, (G * H) // 16, _znum, 0)
        for i in range(G // 16):
            denom_v[pl.ds(i * 16, 16)] = zf
        for i in range(CAP // 16):
            hit_src[pl.ds(i * 16, 16)] = zi

      with jax.named_scope(f"p1_{sidx}"):
        def _p1(i, cnt):
            dvec = e_dst[pl.ds(i * 16, 16)]
            svec = e_src[pl.ds(i * 16, 16)]
            slot = plsc.load_gather(t_mark, [dvec])
            hit = slot >= 0
            slotc = jnp.maximum(slot, 0)
            a_s = plsc.load_gather(t_asrc, [svec], mask=hit)
            a_d = plsc.load_gather(t_adst, [slotc], mask=hit)
            al = a_s + a_d
            al = jnp.where(al >= 0.0, al, 0.2 * al)
            e = jnp.where(hit, jnp.exp(al), 0.0)
            pos = cnt + plsc.cumsum(hit.astype(i32)) - 1
            pos = jnp.minimum(jnp.maximum(pos, 0), CAP - 1)
            plsc.store_scatter(hit_src, [pos], svec, mask=hit)
            plsc.store_scatter(hit_slot, [pos], slot, mask=hit)
            plsc.store_scatter(hit_e, [pos], e, mask=hit)
            return cnt + plsc.all_reduce_population_count(hit)
        cnt = lax.fori_loop(0, NV, _p1, jnp.zeros((16,), i32))
        nh = jnp.minimum(jnp.max(cnt), CAP)

      with jax.named_scope(f"p2_{sidx}"):
        def _chunk(c, carry):
            bs = c * CHUNK
            with jax.named_scope(f"dma{sidx}"):
                for i in range(CHUNK // 16):
                    idx_stage[pl.ds(i * 16, 16)] = \
                        hit_src[pl.ds(bs + i * 16, 16)]
                cps = [
                    pltpu.async_copy(
                        xR.at[idx_stage.at[pl.ds(k * 8, 8)]],
                        rowbuf.at[pl.ds(k * 8, 8)], sem)
                    for k in range(CHUNK // 8)
                ]
                for cp in cps:
                    cp.wait()
            nthis = jnp.minimum(nh - bs, CHUNK)

            def _ph(r, carry2):
                slot_s = hit_slot[pl.ds(bs + r, 16)][0]
                e_bc = jnp.full((16,), hit_e[pl.ds(bs + r, 16)][0],
                                jnp.float32)
                nbase = slot_s * H
                for f in range(H // 16):
                    row = rowbuf[r, pl.ds(f * 16, 16)]
                    plsc.addupdate(numer_v.at[pl.ds(nbase + f * 16, 16)],
                                   e_bc * row)
                plsc.addupdate_scatter(denom_v,
                                       [jnp.full((16,), slot_s, i32)],
                                       e_bc, mask=lane0)
                return carry2
            lax.fori_loop(0, nthis, _ph, 0)
            return carry
        lax.fori_loop(0, (nh + CHUNK - 1) // CHUNK, _chunk, 0)

      with jax.named_scope(f"out{sidx}"):
        pltpu.sync_copy(numer_v, onumR.at[wid])
        pltpu.sync_copy(denom_v, odenR.at[wid])


_k2 = functools.partial(
    pl.kernel,
    mesh=plsc.VectorSubcoreMesh(core_axis_name="c", subcore_axis_name="s",
                                num_cores=NC, num_subcores=NS),
    compiler_params=pltpu.CompilerParams(needs_layout_passes=False),
    out_type=(
        jax.ShapeDtypeStruct((NW, G * H), jnp.float32),
        jax.ShapeDtypeStruct((NW, G), jnp.float32),
        jax.ShapeDtypeStruct((NW, G * H), jnp.float32),
        jax.ShapeDtypeStruct((NW, G), jnp.float32),
    ),
    scratch_types=[
        pltpu.VMEM((EPW,), jnp.int32),
        pltpu.VMEM((EPW,), jnp.int32),
        pltpu.VMEM((N,), jnp.float32),
        pltpu.VMEM((MPAD,), jnp.int32),
        pltpu.VMEM((G,), jnp.float32),
        pltpu.VMEM((CAP + 16,), jnp.int32),
        pltpu.VMEM((CAP + 16,), jnp.int32),
        pltpu.VMEM((CAP + 16,), jnp.float32),
        pltpu.VMEM((CHUNK,), jnp.int32),
        pltpu.VMEM((CHUNK, D), jnp.float32),
        pltpu.VMEM((G * H,), jnp.float32),
        pltpu.VMEM((G,), jnp.float32),
        pltpu.SemaphoreType.DMA,
    ],
)(_k2_body)


def _k3_body(nu_ref, du_ref, sohu_ref, wu_ref, bu_ref,
             nd_ref, dd_ref, sohd_ref, wd_ref, bd_ref,
             mw_ref, mb_ref, o_ref):
    f32 = jnp.float32

    def side(n_ref, d_ref, soh_ref, w_ref, b_ref):
        nsum = jnp.sum(n_ref[...], axis=0)
        dsum = lax.dot_general(d_ref[...], jnp.ones((NW, 1), f32),
                               (((0,), (0,)), ((), ())),
                               preferred_element_type=f32)
        xacc = nsum / (dsum + 1e-16)
        out = jnp.dot(xacc, w_ref[...], preferred_element_type=f32) + b_ref[...]
        emb = jax.nn.sigmoid(out)
        return jnp.dot(soh_ref[...], emb, preferred_element_type=f32)

    r = (side(nu_ref, du_ref, sohu_ref, wu_ref, bu_ref)
         + side(nd_ref, dd_ref, sohd_ref, wd_ref, bd_ref))
    o_ref[...] = jnp.dot(r, mw_ref[...], preferred_element_type=f32) + mb_ref[...]


_k3 = pl.pallas_call(
    _k3_body,
    out_shape=jax.ShapeDtypeStruct((G, 1), jnp.float32),
)


def kernel(up_x, up_edge_index, up_batch, down_x, down_edge_index, down_batch,
           W_up, att_src_up, att_dst_up, bias_up,
           W_down, att_src_down, att_dst_down, bias_down,
           mlp_W, mlp_b):
    i32 = jnp.int32
    pad = EPAD - E

    def split_edges(ei):
        src = jnp.concatenate([ei[0].astype(i32), jnp.zeros((pad,), i32)])
        dst = jnp.concatenate([ei[1].astype(i32),
                               jnp.full((pad,), PADDST, i32)])
        return src, dst

    us, ud = split_edges(up_edge_index)
    ds_, dd = split_edges(down_edge_index)

    asr_u, ads_u, mark_u, soh_u = _k1(
        up_x, W_up, att_src_up.reshape(H, 1), att_dst_up.reshape(H, 1),
        up_batch.astype(i32).reshape(1, N))
    asr_d, ads_d, mark_d, soh_d = _k1(
        down_x, W_down, att_src_down.reshape(H, 1),
        att_dst_down.reshape(H, 1), down_batch.astype(i32).reshape(1, N))

    n_u, d_u, n_d, d_d = _k2(
        us, ud, ds_, dd,
        asr_u.reshape(N), asr_d.reshape(N),
        ads_u.reshape(G), ads_d.reshape(G),
        mark_u.reshape(MPAD), mark_d.reshape(MPAD),
        up_x, down_x)

    return _k3(
        n_u.reshape(NW, G, H), d_u, soh_u, W_up, bias_up.reshape(1, H),
        n_d.reshape(NW, G, H), d_d, soh_d, W_down, bias_down.reshape(1, H),
        mlp_W, mlp_b.reshape(1, 1))


# speedup vs baseline: 1.3601x; 1.3601x over previous
import functools

import jax
import jax.numpy as jnp
from jax import lax
from jax.experimental import pallas as pl
from jax.experimental.pallas import tpu as pltpu
from jax.experimental.pallas import tpu_sc as plsc

N = 10000
E = 160000
D = 256
H = 256
G = 64

NC = 2
NS = 16
NW = NC * NS
EPAD = 160256
EPW = EPAD // NW
NV = EPW // 16
MPAD = N + 80
PADDST = MPAD - 1
CAP = 512
GRP = 8


def _k1_body(x_ref, w_ref, asrc_ref, adst_ref, batch_ref,
             o_asrc, o_adst, o_mark, o_soh):
    f32 = jnp.float32
    i32 = jnp.int32
    x = x_ref[...]
    W = w_ref[...]
    wsrc = jnp.dot(W, asrc_ref[...], preferred_element_type=f32)
    wdst = jnp.dot(W, adst_ref[...], preferred_element_type=f32)
    a_src = jnp.dot(x, wsrc, preferred_element_type=f32)
    a_dst = jnp.dot(x, wdst, preferred_element_type=f32)
    o_asrc[...] = a_src

    batch = batch_ref[...]
    gcol = lax.broadcasted_iota(i32, (G, 1), 0)
    eqg = (gcol == batch).astype(f32)
    ones_col = jnp.ones((N, 1), f32)
    counts = jnp.dot(eqg, ones_col, preferred_element_type=f32)
    g_r = lax.broadcasted_iota(i32, (G, G), 0)
    g_c = lax.broadcasted_iota(i32, (G, G), 1)
    lt = (g_c <= g_r).astype(f32)
    cum = jnp.dot(lt, counts, preferred_element_type=f32)
    sel = cum.astype(i32) - 1
    selw = jnp.where(sel < 0, sel + N, sel)

    n2 = lax.broadcasted_iota(i32, (G, MPAD), 1)
    ohsel = (selw == n2).astype(f32)
    eqm = lax.dot_general(ohsel, ohsel, (((1,), (1,)), ((), ())),
                          preferred_element_type=f32)
    slt = (g_c < g_r).astype(f32)
    dup_before = jnp.dot(eqm * slt, jnp.ones((G, 1), f32),
                         preferred_element_type=f32)
    winner = dup_before < 0.5
    sel_mark = jnp.where(winner, selw, N + gcol)
    ohm = (sel_mark == n2).astype(f32)
    jp1 = (gcol + 1).astype(f32)
    markv = lax.dot_general(ohm, jp1, (((0,), (0,)), ((), ())),
                            preferred_element_type=f32)
    o_mark[...] = markv.astype(i32) - 1
    slot_p1 = jnp.dot(ohsel, markv, preferred_element_type=f32)
    slot_of_g = slot_p1.astype(i32) - 1
    o_soh[...] = (slot_of_g == g_c).astype(f32)
    o_adst[...] = jnp.dot(ohm[:, :N], a_dst, preferred_element_type=f32)


_k1 = pl.pallas_call(
    _k1_body,
    out_shape=(
        jax.ShapeDtypeStruct((N, 1), jnp.float32),
        jax.ShapeDtypeStruct((G, 1), jnp.float32),
        jax.ShapeDtypeStruct((MPAD, 1), jnp.int32),
        jax.ShapeDtypeStruct((G, G), jnp.float32),
    ),
)


def _k2_body(src_up, dst_up, src_dn, dst_dn, asrc_up, asrc_dn,
             adst_up, adst_dn, mark_up, mark_dn, x_up, x_dn,
             o_num_up, o_den_up, o_num_dn, o_den_dn,
             e_src, e_dst, t_asrc, t_mark, t_adst,
             hit_src, hit_slot, hit_e, rowbuf, numer_v, denom_v,
             sem):
    i32 = jnp.int32
    f32 = jnp.float32
    cid = lax.axis_index("c")
    sid = lax.axis_index("s")
    wid = sid * NC + cid
    base_e = wid * EPW
    iota16 = lax.iota(i32, 16)
    zf = jnp.zeros((16,), f32)
    zi = jnp.zeros((16,), i32)
    lane0 = iota16 == 0

    sides = (
        (src_up, dst_up, asrc_up, adst_up, mark_up, x_up, o_num_up, o_den_up),
        (src_dn, dst_dn, asrc_dn, adst_dn, mark_dn, x_dn, o_num_dn, o_den_dn),
    )
    for sidx, (srcR, dstR, asrcR, adstR, markR, xR, onumR, odenR) in enumerate(sides):
      with jax.named_scope(f"stage{sidx}"):
        pltpu.sync_copy(srcR.at[pl.ds(base_e, EPW)], e_src)
        pltpu.sync_copy(dstR.at[pl.ds(base_e, EPW)], e_dst)
        pltpu.sync_copy(asrcR, t_asrc)
        pltpu.sync_copy(markR, t_mark)
        pltpu.sync_copy(adstR, t_adst)

        def _znum(i, c):
            numer_v[pl.ds(i * 16, 16)] = zf
            return c
        lax.fori_loop(0, (G * H) // 16, _znum, 0)
        for i in range(G // 16):
            denom_v[pl.ds(i * 16, 16)] = zf
        for i in range(CAP // 16):
            hit_src[pl.ds(i * 16, 16)] = zi

      with jax.named_scope(f"p1_{sidx}"):
        def _p1(i, cnt):
            dvec = e_dst[pl.ds(i * 16, 16)]
            svec = e_src[pl.ds(i * 16, 16)]
            slot = plsc.load_gather(t_mark, [dvec])
            hit = slot >= 0
            slotc = jnp.maximum(slot, 0)
            a_s = plsc.load_gather(t_asrc, [svec], mask=hit)
            a_d = plsc.load_gather(t_adst, [slotc], mask=hit)
            al = a_s + a_d
            al = jnp.where(al >= 0.0, al, 0.2 * al)
            e = jnp.where(hit, jnp.exp(al), 0.0)
            pos = cnt + plsc.cumsum(hit.astype(i32)) - 1
            pos = jnp.minimum(jnp.maximum(pos, 0), CAP - 1)
            plsc.store_scatter(hit_src, [pos], svec, mask=hit)
            plsc.store_scatter(hit_slot, [pos], slot, mask=hit)
            plsc.store_scatter(hit_e, [pos], e, mask=hit)
            return cnt + plsc.all_reduce_population_count(hit)
        cnt = lax.fori_loop(0, NV, _p1, jnp.zeros((16,), i32))
        nh = jnp.minimum(jnp.max(cnt), CAP)

      with jax.named_scope(f"p2_{sidx}"):
        def _grp(g, carry):
            gb = g * GRP
            with jax.named_scope(f"dma{sidx}"):
                cps = [
                    pltpu.async_copy(
                        xR.at[hit_src[pl.ds(gb + k, 16)][0]],
                        rowbuf.at[k], sem)
                    for k in range(GRP)
                ]
                for cp in cps:
                    cp.wait()
            nthis = jnp.minimum(nh - gb, GRP)

            def _ph(r, carry2):
                slot_s = hit_slot[pl.ds(gb + r, 16)][0]
                e_bc = jnp.full((16,), hit_e[pl.ds(gb + r, 16)][0],
                                jnp.float32)
                nbase = slot_s * H
                for f in range(H // 16):
                    row = rowbuf[r, pl.ds(f * 16, 16)]
                    plsc.addupdate(numer_v.at[pl.ds(nbase + f * 16, 16)],
                                   e_bc * row)
                plsc.addupdate_scatter(denom_v,
                                       [jnp.full((16,), slot_s, i32)],
                                       e_bc, mask=lane0)
                return carry2
            lax.fori_loop(0, nthis, _ph, 0)
            return carry
        lax.fori_loop(0, (nh + GRP - 1) // GRP, _grp, 0)

      with jax.named_scope(f"out{sidx}"):
        pltpu.sync_copy(numer_v, onumR.at[wid])
        pltpu.sync_copy(denom_v, odenR.at[wid])


_k2 = functools.partial(
    pl.kernel,
    mesh=plsc.VectorSubcoreMesh(core_axis_name="c", subcore_axis_name="s",
                                num_cores=NC, num_subcores=NS),
    compiler_params=pltpu.CompilerParams(needs_layout_passes=False),
    out_type=(
        jax.ShapeDtypeStruct((NW, G * H), jnp.float32),
        jax.ShapeDtypeStruct((NW, G), jnp.float32),
        jax.ShapeDtypeStruct((NW, G * H), jnp.float32),
        jax.ShapeDtypeStruct((NW, G), jnp.float32),
    ),
    scratch_types=[
        pltpu.VMEM((EPW,), jnp.int32),
        pltpu.VMEM((EPW,), jnp.int32),
        pltpu.VMEM((N,), jnp.float32),
        pltpu.VMEM((MPAD,), jnp.int32),
        pltpu.VMEM((G,), jnp.float32),
        pltpu.VMEM((CAP + 16,), jnp.int32),
        pltpu.VMEM((CAP + 16,), jnp.int32),
        pltpu.VMEM((CAP + 16,), jnp.float32),
        pltpu.VMEM((GRP, D), jnp.float32),
        pltpu.VMEM((G * H,), jnp.float32),
        pltpu.VMEM((G,), jnp.float32),
        pltpu.SemaphoreType.DMA,
    ],
)(_k2_body)


def _k3_body(nu_ref, du_ref, sohu_ref, wu_ref, bu_ref,
             nd_ref, dd_ref, sohd_ref, wd_ref, bd_ref,
             mw_ref, mb_ref, o_ref):
    f32 = jnp.float32

    def side(n_ref, d_ref, soh_ref, w_ref, b_ref):
        nsum = jnp.sum(n_ref[...], axis=0)
        dsum = lax.dot_general(d_ref[...], jnp.ones((NW, 1), f32),
                               (((0,), (0,)), ((), ())),
                               preferred_element_type=f32)
        xacc = nsum / (dsum + 1e-16)
        out = jnp.dot(xacc, w_ref[...], preferred_element_type=f32) + b_ref[...]
        emb = jax.nn.sigmoid(out)
        return jnp.dot(soh_ref[...], emb, preferred_element_type=f32)

    r = (side(nu_ref, du_ref, sohu_ref, wu_ref, bu_ref)
         + side(nd_ref, dd_ref, sohd_ref, wd_ref, bd_ref))
    o_ref[...] = jnp.dot(r, mw_ref[...], preferred_element_type=f32) + mb_ref[...]


_k3 = pl.pallas_call(
    _k3_body,
    out_shape=jax.ShapeDtypeStruct((G, 1), jnp.float32),
)


def kernel(up_x, up_edge_index, up_batch, down_x, down_edge_index, down_batch,
           W_up, att_src_up, att_dst_up, bias_up,
           W_down, att_src_down, att_dst_down, bias_down,
           mlp_W, mlp_b):
    i32 = jnp.int32
    pad = EPAD - E

    def split_edges(ei):
        src = jnp.concatenate([ei[0].astype(i32), jnp.zeros((pad,), i32)])
        dst = jnp.concatenate([ei[1].astype(i32),
                               jnp.full((pad,), PADDST, i32)])
        return src, dst

    us, ud = split_edges(up_edge_index)
    ds_, dd = split_edges(down_edge_index)

    asr_u, ads_u, mark_u, soh_u = _k1(
        up_x, W_up, att_src_up.reshape(H, 1), att_dst_up.reshape(H, 1),
        up_batch.astype(i32).reshape(1, N))
    asr_d, ads_d, mark_d, soh_d = _k1(
        down_x, W_down, att_src_down.reshape(H, 1),
        att_dst_down.reshape(H, 1), down_batch.astype(i32).reshape(1, N))

    n_u, d_u, n_d, d_d = _k2(
        us, ud, ds_, dd,
        asr_u.reshape(N), asr_d.reshape(N),
        ads_u.reshape(G), ads_d.reshape(G),
        mark_u.reshape(MPAD), mark_d.reshape(MPAD),
        up_x, down_x)

    return _k3(
        n_u.reshape(NW, G, H), d_u, soh_u, W_up, bias_up.reshape(1, H),
        n_d.reshape(NW, G, H), d_d, soh_d, W_down, bias_down.reshape(1, H),
        mlp_W, mlp_b.reshape(1, 1))


# speedup vs baseline: 1.5242x; 1.1207x over previous
import functools

import jax
import jax.numpy as jnp
from jax import lax
from jax.experimental import pallas as pl
from jax.experimental.pallas import tpu as pltpu
from jax.experimental.pallas import tpu_sc as plsc

N = 10000
E = 160000
D = 256
H = 256
G = 64

NC = 2
NS = 16
NW = NC * NS
EREAL = E // NW
EPW = 5008
NV = EPW // 16
MPAD = N + 80
PADDST = MPAD - 1
CAP = 512
GRP = 8


def _k1_side(x, W, av, bv, batch, o_asrc, o_adst, o_mark, o_soh):
    f32 = jnp.float32
    i32 = jnp.int32
    wsrc = jnp.dot(W, av, preferred_element_type=f32)
    wdst = jnp.dot(W, bv, preferred_element_type=f32)
    a_src = jnp.dot(x, wsrc, preferred_element_type=f32)
    a_dst = jnp.dot(x, wdst, preferred_element_type=f32)
    o_asrc[...] = a_src

    gcol = lax.broadcasted_iota(i32, (G, 1), 0)
    eqg = (gcol == batch).astype(f32)
    ones_col = jnp.ones((N, 1), f32)
    counts = jnp.dot(eqg, ones_col, preferred_element_type=f32)
    g_r = lax.broadcasted_iota(i32, (G, G), 0)
    g_c = lax.broadcasted_iota(i32, (G, G), 1)
    lt = (g_c <= g_r).astype(f32)
    cum = jnp.dot(lt, counts, preferred_element_type=f32)
    sel = cum.astype(i32) - 1
    selw = jnp.where(sel < 0, sel + N, sel)

    n2 = lax.broadcasted_iota(i32, (G, MPAD), 1)
    ohsel = (selw == n2).astype(f32)
    eqm = lax.dot_general(ohsel, ohsel, (((1,), (1,)), ((), ())),
                          preferred_element_type=f32)
    slt = (g_c < g_r).astype(f32)
    dup_before = jnp.dot(eqm * slt, jnp.ones((G, 1), f32),
                         preferred_element_type=f32)
    winner = dup_before < 0.5
    sel_mark = jnp.where(winner, selw, N + gcol)
    ohm = (sel_mark == n2).astype(f32)
    jp1 = (gcol + 1).astype(f32)
    markv = lax.dot_general(ohm, jp1, (((0,), (0,)), ((), ())),
                            preferred_element_type=f32)
    o_mark[...] = markv.astype(i32) - 1
    slot_p1 = jnp.dot(ohsel, markv, preferred_element_type=f32)
    slot_of_g = slot_p1.astype(i32) - 1
    o_soh[...] = (slot_of_g == g_c).astype(f32)
    o_adst[...] = jnp.dot(ohm[:, :N], a_dst, preferred_element_type=f32)


def _k1_body(xu_ref, wu_ref, au_ref, bu_ref, batu_ref,
             xd_ref, wd_ref, ad_ref, bd_ref, batd_ref,
             o_asrc_u, o_adst_u, o_mark_u, o_soh_u,
             o_asrc_d, o_adst_d, o_mark_d, o_soh_d):
    _k1_side(xu_ref[...], wu_ref[...], au_ref[...], bu_ref[...],
             batu_ref[...], o_asrc_u, o_adst_u, o_mark_u, o_soh_u)
    _k1_side(xd_ref[...], wd_ref[...], ad_ref[...], bd_ref[...],
             batd_ref[...], o_asrc_d, o_adst_d, o_mark_d, o_soh_d)


_k1 = pl.pallas_call(
    _k1_body,
    out_shape=(
        jax.ShapeDtypeStruct((N, 1), jnp.float32),
        jax.ShapeDtypeStruct((G, 1), jnp.float32),
        jax.ShapeDtypeStruct((MPAD, 1), jnp.int32),
        jax.ShapeDtypeStruct((G, G), jnp.float32),
        jax.ShapeDtypeStruct((N, 1), jnp.float32),
        jax.ShapeDtypeStruct((G, 1), jnp.float32),
        jax.ShapeDtypeStruct((MPAD, 1), jnp.int32),
        jax.ShapeDtypeStruct((G, G), jnp.float32),
    ),
)


def _k2_body(src_up, dst_up, src_dn, dst_dn, asrc_up, asrc_dn,
             adst_up, adst_dn, mark_up, mark_dn, x_up, x_dn,
             o_num_up, o_den_up, o_num_dn, o_den_dn,
             e_src, e_dst, t_asrc, t_mark, t_adst,
             hit_src, hit_slot, hit_e, rowbuf, numer_v, denom_v,
             sem):
    i32 = jnp.int32
    f32 = jnp.float32
    cid = lax.axis_index("c")
    sid = lax.axis_index("s")
    wid = sid * NC + cid
    base_e = wid * EREAL
    iota16 = lax.iota(i32, 16)
    zf = jnp.zeros((16,), f32)
    zi = jnp.zeros((16,), i32)
    lane0 = iota16 == 0
    tailmask = iota16 >= (EREAL - (NV - 1) * 16)

    sides = (
        (src_up, dst_up, asrc_up, adst_up, mark_up, x_up, o_num_up, o_den_up),
        (src_dn, dst_dn, asrc_dn, adst_dn, mark_dn, x_dn, o_num_dn, o_den_dn),
    )
    for sidx, (srcR, dstR, asrcR, adstR, markR, xR, onumR, odenR) in enumerate(sides):
      with jax.named_scope(f"stage{sidx}"):
        cps = [
            pltpu.async_copy(srcR.at[pl.ds(base_e, EREAL)],
                             e_src.at[pl.ds(0, EREAL)], sem),
            pltpu.async_copy(dstR.at[pl.ds(base_e, EREAL)],
                             e_dst.at[pl.ds(0, EREAL)], sem),
            pltpu.async_copy(asrcR, t_asrc, sem),
            pltpu.async_copy(markR, t_mark, sem),
            pltpu.async_copy(adstR, t_adst, sem),
        ]

        def _znum(i, c):
            numer_v[pl.ds(i * 16, 16)] = zf
            return c
        lax.fori_loop(0, (G * H) // 16, _znum, 0)
        for i in range(G // 16):
            denom_v[pl.ds(i * 16, 16)] = zf
        for i in range(CAP // 16):
            hit_src[pl.ds(i * 16, 16)] = zi
        for cp in cps:
            cp.wait()
        tb = (NV - 1) * 16
        plsc.store_scatter(e_src, [iota16 + tb], zi, mask=tailmask)
        plsc.store_scatter(e_dst, [iota16 + tb],
                           jnp.full((16,), PADDST, i32), mask=tailmask)

      with jax.named_scope(f"p1_{sidx}"):
        def _p1(i, cnt):
            dvec = e_dst[pl.ds(i * 16, 16)]
            svec = e_src[pl.ds(i * 16, 16)]
            slot = plsc.load_gather(t_mark, [dvec])
            hit = slot >= 0
            slotc = jnp.maximum(slot, 0)
            a_s = plsc.load_gather(t_asrc, [svec], mask=hit)
            a_d = plsc.load_gather(t_adst, [slotc], mask=hit)
            al = a_s + a_d
            al = jnp.where(al >= 0.0, al, 0.2 * al)
            e = jnp.where(hit, jnp.exp(al), 0.0)
            pos = cnt + plsc.cumsum(hit.astype(i32)) - 1
            pos = jnp.minimum(jnp.maximum(pos, 0), CAP - 1)
            plsc.store_scatter(hit_src, [pos], svec, mask=hit)
            plsc.store_scatter(hit_slot, [pos], slot, mask=hit)
            plsc.store_scatter(hit_e, [pos], e, mask=hit)
            return cnt + plsc.all_reduce_population_count(hit)
        cnt = lax.fori_loop(0, NV, _p1, jnp.zeros((16,), i32))
        nh = jnp.minimum(jnp.max(cnt), CAP)

      with jax.named_scope(f"p2_{sidx}"):
        def _grp(g, carry):
            gb = g * GRP
            with jax.named_scope(f"dma{sidx}"):
                cps = [
                    pltpu.async_copy(
                        xR.at[hit_src[pl.ds(gb + k, 16)][0]],
                        rowbuf.at[k], sem)
                    for k in range(GRP)
                ]
                for cp in cps:
                    cp.wait()
            nthis = jnp.minimum(nh - gb, GRP)

            def _ph(r, carry2):
                slot_s = hit_slot[pl.ds(gb + r, 16)][0]
                e_bc = jnp.full((16,), hit_e[pl.ds(gb + r, 16)][0],
                                jnp.float32)
                nbase = slot_s * H
                for f in range(H // 16):
                    row = rowbuf[r, pl.ds(f * 16, 16)]
                    plsc.addupdate(numer_v.at[pl.ds(nbase + f * 16, 16)],
                                   e_bc * row)
                plsc.addupdate_scatter(denom_v,
                                       [jnp.full((16,), slot_s, i32)],
                                       e_bc, mask=lane0)
                return carry2
            lax.fori_loop(0, nthis, _ph, 0)
            return carry
        lax.fori_loop(0, (nh + GRP - 1) // GRP, _grp, 0)

      with jax.named_scope(f"out{sidx}"):
        pltpu.sync_copy(numer_v, onumR.at[wid])
        pltpu.sync_copy(denom_v, odenR.at[wid])


_k2 = functools.partial(
    pl.kernel,
    mesh=plsc.VectorSubcoreMesh(core_axis_name="c", subcore_axis_name="s",
                                num_cores=NC, num_subcores=NS),
    compiler_params=pltpu.CompilerParams(needs_layout_passes=False),
    out_type=(
        jax.ShapeDtypeStruct((NW, G * H), jnp.float32),
        jax.ShapeDtypeStruct((NW, G), jnp.float32),
        jax.ShapeDtypeStruct((NW, G * H), jnp.float32),
        jax.ShapeDtypeStruct((NW, G), jnp.float32),
    ),
    scratch_types=[
        pltpu.VMEM((EPW,), jnp.int32),
        pltpu.VMEM((EPW,), jnp.int32),
        pltpu.VMEM((N,), jnp.float32),
        pltpu.VMEM((MPAD,), jnp.int32),
        pltpu.VMEM((G,), jnp.float32),
        pltpu.VMEM((CAP + 16,), jnp.int32),
        pltpu.VMEM((CAP + 16,), jnp.int32),
        pltpu.VMEM((CAP + 16,), jnp.float32),
        pltpu.VMEM((GRP, D), jnp.float32),
        pltpu.VMEM((G * H,), jnp.float32),
        pltpu.VMEM((G,), jnp.float32),
        pltpu.SemaphoreType.DMA,
    ],
)(_k2_body)


def _k3_body(nu_ref, du_ref, sohu_ref, wu_ref, bu_ref,
             nd_ref, dd_ref, sohd_ref, wd_ref, bd_ref,
             mw_ref, mb_ref, o_ref):
    f32 = jnp.float32

    def side(n_ref, d_ref, soh_ref, w_ref, b_ref):
        nsum = jnp.sum(n_ref[...], axis=0)
        dsum = lax.dot_general(d_ref[...], jnp.ones((NW, 1), f32),
                               (((0,), (0,)), ((), ())),
                               preferred_element_type=f32)
        xacc = nsum / (dsum + 1e-16)
        out = jnp.dot(xacc, w_ref[...], preferred_element_type=f32) + b_ref[...]
        emb = jax.nn.sigmoid(out)
        return jnp.dot(soh_ref[...], emb, preferred_element_type=f32)

    r = (side(nu_ref, du_ref, sohu_ref, wu_ref, bu_ref)
         + side(nd_ref, dd_ref, sohd_ref, wd_ref, bd_ref))
    o_ref[...] = jnp.dot(r, mw_ref[...], preferred_element_type=f32) + mb_ref[...]


_k3 = pl.pallas_call(
    _k3_body,
    out_shape=jax.ShapeDtypeStruct((G, 1), jnp.float32),
)


def kernel(up_x, up_edge_index, up_batch, down_x, down_edge_index, down_batch,
           W_up, att_src_up, att_dst_up, bias_up,
           W_down, att_src_down, att_dst_down, bias_down,
           mlp_W, mlp_b):
    i32 = jnp.int32

    (asr_u, ads_u, mark_u, soh_u,
     asr_d, ads_d, mark_d, soh_d) = _k1(
        up_x, W_up, att_src_up.reshape(H, 1), att_dst_up.reshape(H, 1),
        up_batch.astype(i32).reshape(1, N),
        down_x, W_down, att_src_down.reshape(H, 1),
        att_dst_down.reshape(H, 1), down_batch.astype(i32).reshape(1, N))

    n_u, d_u, n_d, d_d = _k2(
        up_edge_index[0].astype(i32), up_edge_index[1].astype(i32),
        down_edge_index[0].astype(i32), down_edge_index[1].astype(i32),
        asr_u.reshape(N), asr_d.reshape(N),
        ads_u.reshape(G), ads_d.reshape(G),
        mark_u.reshape(MPAD), mark_d.reshape(MPAD),
        up_x, down_x)

    return _k3(
        n_u.reshape(NW, G, H), d_u, soh_u, W_up, bias_up.reshape(1, H),
        n_d.reshape(NW, G, H), d_d, soh_d, W_down, bias_down.reshape(1, H),
        mlp_W, mlp_b.reshape(1, 1))
